# baseline (speedup 1.0000x reference)
import functools

import jax
import jax.numpy as jnp
from jax import lax
from jax.experimental import pallas as pl
from jax.experimental.pallas import tpu as pltpu
from jax.experimental.pallas import tpu_sc as plsc

F32 = jnp.float32
EPS = 1e-5

NC = 2
NS = 16
NW = NC * NS
EB = 80
KB = 3


def _sc_mesh():
    return plsc.VectorSubcoreMesh(core_axis_name="c", subcore_axis_name="s")


def _row_split(n):
    r0 = (-(-n // NS) + 7) // 8 * 8
    rlast = n - (NS - 1) * r0
    assert 0 < rlast <= r0
    return r0, rlast


def _tilewise_copy(s, n, copy_fn):
    r0, rlast = _row_split(n)

    @pl.when(s < NS - 1)
    def _():
        copy_fn(s * r0, r0)

    @pl.when(s == NS - 1)
    def _():
        copy_fn((NS - 1) * r0, rlast)


def _make_sc_deg(n, nb):
    @functools.partial(
        pl.kernel,
        mesh=_sc_mesh(),
        out_type=jax.ShapeDtypeStruct((NC, n, 16), F32),
        compiler_params=pltpu.CompilerParams(use_tc_tiling_on_sc=False),
        scratch_types=[
            pltpu.VMEM((nb, EB), jnp.int32),
            pltpu.VMEM((EB, 16), F32),
            pltpu.VMEM_SHARED((n, 16), F32),
        ],
    )
    def sc_deg(dst_hbm, ones_hbm, zeros_hbm, degp_hbm, idx_v, ones_v, dacc):
        c = lax.axis_index("c")
        s = lax.axis_index("s")
        w = c * NS + s
        pltpu.sync_copy(dst_hbm.at[w], idx_v)
        pltpu.sync_copy(ones_hbm, ones_v)
        _tilewise_copy(s, n, lambda off, sz: pltpu.sync_copy(
            zeros_hbm.at[pl.ds(0, sz)], dacc.at[pl.ds(off, sz)]))
        plsc.subcore_barrier()

        def body(j, carry):
            pltpu.sync_copy(ones_v, dacc.at[idx_v.at[j]], add=True)
            return carry

        lax.fori_loop(0, nb, body, 0)
        plsc.subcore_barrier()
        _tilewise_copy(s, n, lambda off, sz: pltpu.sync_copy(
            dacc.at[pl.ds(off, sz)], degp_hbm.at[c, pl.ds(off, sz)]))

    return sc_deg


def _make_sc_agg(n, d, nb):
    assert nb % KB == 0 and nb >= 2 * KB

    @functools.partial(
        pl.kernel,
        mesh=_sc_mesh(),
        out_type=jax.ShapeDtypeStruct((NC, n, d), F32),
        compiler_params=pltpu.CompilerParams(use_tc_tiling_on_sc=False),
        scratch_types=(
            [pltpu.VMEM((nb, EB), jnp.int32),
             pltpu.VMEM((nb, EB), jnp.int32)]
            + [pltpu.VMEM((EB, d), F32) for _ in range(KB)]
            + [pltpu.VMEM_SHARED((n, d), F32)]
            + [pltpu.SemaphoreType.DMA for _ in range(KB)]
        ),
    )
    def sc_agg(yw_hbm, src_hbm, dst_hbm, out_hbm, isrc, idst, *rest):
        rows = rest[:KB]
        acc = rest[KB]
        sems = rest[KB + 1:]
        c = lax.axis_index("c")
        s = lax.axis_index("s")
        w = c * NS + s
        pltpu.sync_copy(src_hbm.at[w], isrc)
        pltpu.sync_copy(dst_hbm.at[w], idst)
        _tilewise_copy(s, n, lambda off, sz: pltpu.sync_copy(
            yw_hbm.at[pl.ds(off, sz)], acc.at[pl.ds(off, sz)]))
        plsc.subcore_barrier()

        for b in range(KB):
            pltpu.async_copy(yw_hbm.at[isrc.at[b]], rows[b], sems[b])

        def body(t, carry):
            for b in range(KB):
                j = KB * t + b
                jn = lax.min(j + KB, nb - 1)
                pltpu.make_async_copy(
                    yw_hbm.at[isrc.at[j]], rows[b], sems[b]).wait()
                pltpu.sync_copy(rows[b], acc.at[idst.at[j]], add=True)
                pltpu.async_copy(yw_hbm.at[isrc.at[jn]], rows[b], sems[b])
            return carry

        lax.fori_loop(0, nb // KB, body, 0)
        for b in range(KB):
            pltpu.make_async_copy(yw_hbm.at[isrc.at[0]], rows[b],
                                  sems[b]).wait()
        plsc.subcore_barrier()
        _tilewise_copy(s, n, lambda off, sz: pltpu.sync_copy(
            acc.at[pl.ds(off, sz)], out_hbm.at[c, pl.ds(off, sz)]))

    return sc_agg


def _row_mask(n, npad, width):
    rows = lax.broadcasted_iota(jnp.int32, (n, width), 0)
    return jnp.where(rows < npad, 1.0, 0.0).astype(F32)


def _make_tc1(n, npad):
    def _tc1_body(x_ref, w1_ref, degp_ref, yw_ref, dinv_ref):
        deg = degp_ref[0] + degp_ref[1] + (1.0 - _row_mask(n, npad, 16))
        dinv = lax.rsqrt(deg)
        dinv_ref[...] = dinv
        xw = jnp.dot(x_ref[...], w1_ref[...], preferred_element_type=F32)
        yw_ref[...] = xw * dinv[:, 0:1]

    return _tc1_body


def _make_tc2(n, npad):
    def _tc2_body(p_ref, yw_ref, dinv_ref, gamma_ref, beta_ref, b1_ref,
                  w2_ref, zw_ref):
        dinv1 = dinv_ref[:, 0:1]
        ywc = yw_ref[...] * (1.0 + _row_mask(n, npad, 1))
        h = dinv1 * (p_ref[0] + p_ref[1] - ywc) + b1_ref[...]
        mean = jnp.mean(h, axis=0, keepdims=True)
        var = jnp.mean((h - mean) ** 2, axis=0, keepdims=True)
        hn = (h - mean) * lax.rsqrt(var + EPS) * gamma_ref[...] + beta_ref[...]
        hrelu = jnp.maximum(hn, 0.0)
        hw = jnp.dot(hrelu, w2_ref[...], preferred_element_type=F32)
        zw_ref[...] = hw * dinv1

    return _tc2_body


def _make_tc3(n, npad):
    def _tc3_body(p_ref, zw_ref, dinv_ref, b2_ref, out_ref):
        dinv1 = dinv_ref[:, 0:1]
        zwc = zw_ref[...] * (1.0 + _row_mask(n, npad, 1))
        out_ref[...] = dinv1 * (p_ref[0] + p_ref[1] - zwc) + b2_ref[...]

    return _tc3_body


def kernel(x, edge_index, W1, b1, gamma, beta, W2, b2):
    n, _ = x.shape
    e = edge_index.shape[1]
    d1 = W1.shape[1]
    d2 = W2.shape[1]

    nb = -(-e // (NW * EB))
    nb = -(-nb // KB) * KB
    ep = nb * NW * EB
    npad = ep - e
    assert npad <= n

    pad = jnp.arange(npad, dtype=edge_index.dtype)
    ei = jnp.concatenate([edge_index, jnp.stack([pad, pad])], axis=1)
    src3 = ei[0].reshape(NW, nb, EB)
    dst3 = ei[1].reshape(NW, nb, EB)
    ones16 = jnp.ones((EB, 16), F32)
    zeros16 = jnp.zeros((_row_split(n)[0], 16), F32)

    degp = _make_sc_deg(n, nb)(dst3, ones16, zeros16)

    yw, dinv = pl.pallas_call(
        _make_tc1(n, npad),
        out_shape=(jax.ShapeDtypeStruct((n, d1), F32),
                   jax.ShapeDtypeStruct((n, 16), F32)),
    )(x, W1, degp)

    p1 = _make_sc_agg(n, d1, nb)(yw, src3, dst3)

    zw = pl.pallas_call(
        _make_tc2(n, npad),
        out_shape=jax.ShapeDtypeStruct((n, d2), F32),
    )(p1, yw, dinv, gamma.reshape(1, d1), beta.reshape(1, d1),
      b1.reshape(1, d1), W2)

    p2 = _make_sc_agg(n, d2, nb)(zw, src3, dst3)

    out = pl.pallas_call(
        _make_tc3(n, npad),
        out_shape=jax.ShapeDtypeStruct((n, d2), F32),
    )(p2, zw, dinv, b2.reshape(1, d2))

    return out


# speedup vs baseline: 1.0029x; 1.0029x over previous
import functools

import jax
import jax.numpy as jnp
from jax import lax
from jax.experimental import pallas as pl
from jax.experimental.pallas import tpu as pltpu
from jax.experimental.pallas import tpu_sc as plsc

F32 = jnp.float32
EPS = 1e-5

NC = 2
NS = 16
NW = NC * NS
EB = 80
KB = 3


def _sc_mesh():
    return plsc.VectorSubcoreMesh(core_axis_name="c", subcore_axis_name="s")


def _row_split(n):
    r0 = (-(-n // NS) + 7) // 8 * 8
    rlast = n - (NS - 1) * r0
    assert 0 < rlast <= r0
    return r0, rlast


def _tilewise_copy(s, n, copy_fn):
    r0, rlast = _row_split(n)

    @pl.when(s < NS - 1)
    def _():
        copy_fn(s * r0, r0)

    @pl.when(s == NS - 1)
    def _():
        copy_fn((NS - 1) * r0, rlast)


def _make_sc_deg(n, nb):
    @functools.partial(
        pl.kernel,
        mesh=_sc_mesh(),
        out_type=jax.ShapeDtypeStruct((NC, n, 16), F32),
        compiler_params=pltpu.CompilerParams(use_tc_tiling_on_sc=False),
        scratch_types=[
            pltpu.VMEM((nb, EB), jnp.int32),
            pltpu.VMEM((EB, 16), F32),
            pltpu.VMEM_SHARED((n, 16), F32),
        ],
    )
    def sc_deg(dst_hbm, ones_hbm, zeros_hbm, degp_hbm, idx_v, ones_v, dacc):
        c = lax.axis_index("c")
        s = lax.axis_index("s")
        w = c * NS + s
        pltpu.sync_copy(dst_hbm.at[w], idx_v)
        pltpu.sync_copy(ones_hbm, ones_v)
        _tilewise_copy(s, n, lambda off, sz: pltpu.sync_copy(
            zeros_hbm.at[pl.ds(0, sz)], dacc.at[pl.ds(off, sz)]))
        plsc.subcore_barrier()

        def body(j, carry):
            pltpu.sync_copy(ones_v, dacc.at[idx_v.at[j]], add=True)
            return carry

        lax.fori_loop(0, nb, body, 0)
        plsc.subcore_barrier()
        _tilewise_copy(s, n, lambda off, sz: pltpu.sync_copy(
            dacc.at[pl.ds(off, sz)], degp_hbm.at[c, pl.ds(off, sz)]))

    return sc_deg


def _make_sc_agg(n, d, nb):
    assert nb % KB == 0 and nb >= 2 * KB

    @functools.partial(
        pl.kernel,
        mesh=_sc_mesh(),
        out_type=jax.ShapeDtypeStruct((NC, n, d), F32),
        compiler_params=pltpu.CompilerParams(use_tc_tiling_on_sc=False),
        scratch_types=(
            [pltpu.VMEM((nb, EB), jnp.int32),
             pltpu.VMEM((nb, EB), jnp.int32)]
            + [pltpu.VMEM((EB, d), F32) for _ in range(KB)]
            + [pltpu.VMEM_SHARED((n, d), F32)]
            + [pltpu.SemaphoreType.DMA for _ in range(KB)]
        ),
    )
    def sc_agg(yw_hbm, src_hbm, dst_hbm, out_hbm, isrc, idst, *rest):
        rows = rest[:KB]
        acc = rest[KB]
        sems = rest[KB + 1:]
        c = lax.axis_index("c")
        s = lax.axis_index("s")
        w = c * NS + s
        pltpu.sync_copy(src_hbm.at[w], isrc)
        pltpu.sync_copy(dst_hbm.at[w], idst)
        for b in range(KB):
            pltpu.async_copy(yw_hbm.at[isrc.at[b]], rows[b], sems[b])
        _tilewise_copy(s, n, lambda off, sz: pltpu.sync_copy(
            yw_hbm.at[pl.ds(off, sz)], acc.at[pl.ds(off, sz)]))
        plsc.subcore_barrier()

        def body(t, carry):
            for b in range(KB):
                j = KB * t + b
                jn = lax.min(j + KB, nb - 1)
                pltpu.make_async_copy(
                    yw_hbm.at[isrc.at[j]], rows[b], sems[b]).wait()
                pltpu.sync_copy(rows[b], acc.at[idst.at[j]], add=True)
                pltpu.async_copy(yw_hbm.at[isrc.at[jn]], rows[b], sems[b])
            return carry

        lax.fori_loop(0, nb // KB, body, 0)
        for b in range(KB):
            pltpu.make_async_copy(yw_hbm.at[isrc.at[0]], rows[b],
                                  sems[b]).wait()
        plsc.subcore_barrier()
        _tilewise_copy(s, n, lambda off, sz: pltpu.sync_copy(
            acc.at[pl.ds(off, sz)], out_hbm.at[c, pl.ds(off, sz)]))

    return sc_agg


def _row_mask(n, npad, width):
    rows = lax.broadcasted_iota(jnp.int32, (n, width), 0)
    return jnp.where(rows < npad, 1.0, 0.0).astype(F32)


def _tc0_body(x_ref, w1_ref, xw_ref):
    xw_ref[...] = jnp.dot(x_ref[...], w1_ref[...], preferred_element_type=F32)


def _make_tc1(n, npad):
    def _tc1_body(xw_ref, degp_ref, yw_ref, dinv_ref):
        deg = degp_ref[0] + degp_ref[1] + (1.0 - _row_mask(n, npad, 16))
        dinv = lax.rsqrt(deg)
        dinv_ref[...] = dinv
        yw_ref[...] = xw_ref[...] * dinv[:, 0:1]

    return _tc1_body


def _make_tc2(n, npad):
    def _tc2_body(p_ref, yw_ref, dinv_ref, gamma_ref, beta_ref, b1_ref,
                  w2_ref, zw_ref):
        dinv1 = dinv_ref[:, 0:1]
        ywc = yw_ref[...] * (1.0 + _row_mask(n, npad, 1))
        h = dinv1 * (p_ref[0] + p_ref[1] - ywc) + b1_ref[...]
        mean = jnp.mean(h, axis=0, keepdims=True)
        var = jnp.mean((h - mean) ** 2, axis=0, keepdims=True)
        hn = (h - mean) * lax.rsqrt(var + EPS) * gamma_ref[...] + beta_ref[...]
        hrelu = jnp.maximum(hn, 0.0)
        hw = jnp.dot(hrelu, w2_ref[...], preferred_element_type=F32)
        zw_ref[...] = hw * dinv1

    return _tc2_body


def _make_tc3(n, npad):
    def _tc3_body(p_ref, zw_ref, dinv_ref, b2_ref, out_ref):
        dinv1 = dinv_ref[:, 0:1]
        zwc = zw_ref[...] * (1.0 + _row_mask(n, npad, 1))
        out_ref[...] = dinv1 * (p_ref[0] + p_ref[1] - zwc) + b2_ref[...]

    return _tc3_body


def kernel(x, edge_index, W1, b1, gamma, beta, W2, b2):
    n, _ = x.shape
    e = edge_index.shape[1]
    d1 = W1.shape[1]
    d2 = W2.shape[1]

    nb = -(-e // (NW * EB))
    nb = -(-nb // KB) * KB
    ep = nb * NW * EB
    npad = ep - e
    assert npad <= n

    pad = jnp.arange(npad, dtype=edge_index.dtype)
    ei = jnp.concatenate([edge_index, jnp.stack([pad, pad])], axis=1)
    src3 = ei[0].reshape(NW, nb, EB)
    dst3 = ei[1].reshape(NW, nb, EB)
    ones16 = jnp.ones((EB, 16), F32)
    zeros16 = jnp.zeros((_row_split(n)[0], 16), F32)

    xw = pl.pallas_call(
        _tc0_body,
        out_shape=jax.ShapeDtypeStruct((n, d1), F32),
    )(x, W1)

    degp = _make_sc_deg(n, nb)(dst3, ones16, zeros16)

    yw, dinv = pl.pallas_call(
        _make_tc1(n, npad),
        out_shape=(jax.ShapeDtypeStruct((n, d1), F32),
                   jax.ShapeDtypeStruct((n, 16), F32)),
    )(xw, degp)

    p1 = _make_sc_agg(n, d1, nb)(yw, src3, dst3)

    zw = pl.pallas_call(
        _make_tc2(n, npad),
        out_shape=jax.ShapeDtypeStruct((n, d2), F32),
    )(p1, yw, dinv, gamma.reshape(1, d1), beta.reshape(1, d1),
      b1.reshape(1, d1), W2)

    p2 = _make_sc_agg(n, d2, nb)(zw, src3, dst3)

    out = pl.pallas_call(
        _make_tc3(n, npad),
        out_shape=jax.ShapeDtypeStruct((n, d2), F32),
    )(p2, zw, dinv, b2.reshape(1, d2))

    return out


# speedup vs baseline: 1.1986x; 1.1952x over previous
import functools

import jax
import jax.numpy as jnp
from jax import lax
from jax.experimental import pallas as pl
from jax.experimental.pallas import tpu as pltpu
from jax.experimental.pallas import tpu_sc as plsc

F32 = jnp.float32
BF16 = jnp.bfloat16
EPS = 1e-5

NC = 2
NS = 16
NW = NC * NS
EB = 128
KB = 4


def _sc_mesh():
    return plsc.VectorSubcoreMesh(core_axis_name="c", subcore_axis_name="s")


def _row_split(n):
    r0 = (-(-n // NS) + 15) // 16 * 16
    rlast = n - (NS - 1) * r0
    assert 0 < rlast <= r0
    return r0, rlast


def _tilewise_copy(s, n, copy_fn):
    r0, rlast = _row_split(n)

    @pl.when(s < NS - 1)
    def _():
        copy_fn(s * r0, r0)

    @pl.when(s == NS - 1)
    def _():
        copy_fn((NS - 1) * r0, rlast)


def _make_sc_deg(n, nb):
    @functools.partial(
        pl.kernel,
        mesh=_sc_mesh(),
        out_type=jax.ShapeDtypeStruct((NC, n, 16), F32),
        compiler_params=pltpu.CompilerParams(use_tc_tiling_on_sc=False),
        scratch_types=[
            pltpu.VMEM((nb, EB), jnp.int32),
            pltpu.VMEM((EB, 16), F32),
            pltpu.VMEM_SHARED((n, 16), F32),
        ],
    )
    def sc_deg(dst_hbm, ones_hbm, zeros_hbm, degp_hbm, idx_v, ones_v, dacc):
        c = lax.axis_index("c")
        s = lax.axis_index("s")
        w = c * NS + s
        pltpu.sync_copy(dst_hbm.at[w], idx_v)
        pltpu.sync_copy(ones_hbm, ones_v)
        _tilewise_copy(s, n, lambda off, sz: pltpu.sync_copy(
            zeros_hbm.at[pl.ds(0, sz)], dacc.at[pl.ds(off, sz)]))
        plsc.subcore_barrier()

        def body(j, carry):
            pltpu.sync_copy(ones_v, dacc.at[idx_v.at[j]], add=True)
            return carry

        lax.fori_loop(0, nb, body, 0)
        plsc.subcore_barrier()
        _tilewise_copy(s, n, lambda off, sz: pltpu.sync_copy(
            dacc.at[pl.ds(off, sz)], degp_hbm.at[c, pl.ds(off, sz)]))

    return sc_deg


def _make_sc_agg(n, d, nb):
    assert nb % KB == 0 and nb >= 2 * KB

    @functools.partial(
        pl.kernel,
        mesh=_sc_mesh(),
        out_type=jax.ShapeDtypeStruct((NC, n, d), BF16),
        compiler_params=pltpu.CompilerParams(use_tc_tiling_on_sc=False),
        scratch_types=(
            [pltpu.VMEM((nb, EB), jnp.int32),
             pltpu.VMEM((nb, EB), jnp.int32)]
            + [pltpu.VMEM((EB, d), BF16) for _ in range(KB)]
            + [pltpu.VMEM_SHARED((n, d), BF16)]
            + [pltpu.SemaphoreType.DMA for _ in range(KB)]
        ),
    )
    def sc_agg(yw_hbm, src_hbm, dst_hbm, out_hbm, isrc, idst, *rest):
        rows = rest[:KB]
        acc = rest[KB]
        sems = rest[KB + 1:]
        c = lax.axis_index("c")
        s = lax.axis_index("s")
        w = c * NS + s
        pltpu.sync_copy(src_hbm.at[w], isrc)
        pltpu.sync_copy(dst_hbm.at[w], idst)
        for b in range(KB):
            pltpu.async_copy(yw_hbm.at[isrc.at[b]], rows[b], sems[b])
        _tilewise_copy(s, n, lambda off, sz: pltpu.sync_copy(
            yw_hbm.at[pl.ds(off, sz)], acc.at[pl.ds(off, sz)]))
        plsc.subcore_barrier()

        def body(t, carry):
            for b in range(KB):
                j = KB * t + b
                jn = lax.min(j + KB, nb - 1)
                pltpu.make_async_copy(
                    yw_hbm.at[isrc.at[j]], rows[b], sems[b]).wait()
                pltpu.sync_copy(rows[b], acc.at[idst.at[j]], add=True)
                pltpu.async_copy(yw_hbm.at[isrc.at[jn]], rows[b], sems[b])
            return carry

        lax.fori_loop(0, nb // KB, body, 0)
        for b in range(KB):
            pltpu.make_async_copy(yw_hbm.at[isrc.at[0]], rows[b],
                                  sems[b]).wait()
        plsc.subcore_barrier()
        _tilewise_copy(s, n, lambda off, sz: pltpu.sync_copy(
            acc.at[pl.ds(off, sz)], out_hbm.at[c, pl.ds(off, sz)]))

    return sc_agg


def _row_mask(n, npad, width):
    rows = lax.broadcasted_iota(jnp.int32, (n, width), 0)
    return jnp.where(rows < npad, 1.0, 0.0).astype(F32)


def _tc0_body(x_ref, w1_ref, xw_ref):
    xw_ref[...] = jnp.dot(x_ref[...], w1_ref[...], preferred_element_type=F32)


def _make_tc1(n, npad):
    def _tc1_body(xw_ref, degp_ref, yw_ref, dinv_ref):
        deg = degp_ref[0] + degp_ref[1] + (1.0 - _row_mask(n, npad, 16))
        dinv = lax.rsqrt(deg)
        dinv_ref[...] = dinv
        yw_ref[...] = (xw_ref[...] * dinv[:, 0:1]).astype(BF16)

    return _tc1_body


def _make_tc2(n, npad):
    def _tc2_body(p_ref, yw_ref, dinv_ref, gamma_ref, beta_ref, b1_ref,
                  w2_ref, zw_ref):
        dinv1 = dinv_ref[:, 0:1]
        ywc = yw_ref[...].astype(F32) * (1.0 + _row_mask(n, npad, 1))
        p = p_ref[0].astype(F32) + p_ref[1].astype(F32)
        h = dinv1 * (p - ywc) + b1_ref[...]
        mean = jnp.mean(h, axis=0, keepdims=True)
        var = jnp.mean((h - mean) ** 2, axis=0, keepdims=True)
        hn = (h - mean) * lax.rsqrt(var + EPS) * gamma_ref[...] + beta_ref[...]
        hrelu = jnp.maximum(hn, 0.0)
        hw = jnp.dot(hrelu, w2_ref[...], preferred_element_type=F32)
        zw_ref[...] = (hw * dinv1).astype(BF16)

    return _tc2_body


def _make_tc3(n, npad):
    def _tc3_body(p_ref, zw_ref, dinv_ref, b2_ref, out_ref):
        dinv1 = dinv_ref[:, 0:1]
        zwc = zw_ref[...].astype(F32) * (1.0 + _row_mask(n, npad, 1))
        p = p_ref[0].astype(F32) + p_ref[1].astype(F32)
        out_ref[...] = dinv1 * (p - zwc) + b2_ref[...]

    return _tc3_body


def kernel(x, edge_index, W1, b1, gamma, beta, W2, b2):
    n, _ = x.shape
    e = edge_index.shape[1]
    d1 = W1.shape[1]
    d2 = W2.shape[1]

    nb = -(-e // (NW * EB))
    nb = -(-nb // KB) * KB
    ep = nb * NW * EB
    npad = ep - e
    assert npad <= n

    pad = jnp.arange(npad, dtype=edge_index.dtype)
    ei = jnp.concatenate([edge_index, jnp.stack([pad, pad])], axis=1)
    src3 = ei[0].reshape(NW, nb, EB)
    dst3 = ei[1].reshape(NW, nb, EB)
    ones16 = jnp.ones((EB, 16), F32)
    zeros16 = jnp.zeros((_row_split(n)[0], 16), F32)

    xw = pl.pallas_call(
        _tc0_body,
        out_shape=jax.ShapeDtypeStruct((n, d1), F32),
    )(x, W1)

    degp = _make_sc_deg(n, nb)(dst3, ones16, zeros16)

    yw, dinv = pl.pallas_call(
        _make_tc1(n, npad),
        out_shape=(jax.ShapeDtypeStruct((n, d1), BF16),
                   jax.ShapeDtypeStruct((n, 16), F32)),
    )(xw, degp)

    p1 = _make_sc_agg(n, d1, nb)(yw, src3, dst3)

    zw = pl.pallas_call(
        _make_tc2(n, npad),
        out_shape=jax.ShapeDtypeStruct((n, d2), BF16),
    )(p1, yw, dinv, gamma.reshape(1, d1), beta.reshape(1, d1),
      b1.reshape(1, d1), W2)

    p2 = _make_sc_agg(n, d2, nb)(zw, src3, dst3)

    out = pl.pallas_call(
        _make_tc3(n, npad),
        out_shape=jax.ShapeDtypeStruct((n, d2), F32),
    )(p2, zw, dinv, b2.reshape(1, d2))

    return out


# speedup vs baseline: 1.2128x; 1.0118x over previous
import functools

import jax
import jax.numpy as jnp
from jax import lax
from jax.experimental import pallas as pl
from jax.experimental.pallas import tpu as pltpu
from jax.experimental.pallas import tpu_sc as plsc

F32 = jnp.float32
BF16 = jnp.bfloat16
EPS = 1e-5

NC = 2
NS = 16
NW = NC * NS
EB = 128
KB = 5


def _sc_mesh():
    return plsc.VectorSubcoreMesh(core_axis_name="c", subcore_axis_name="s")


def _row_split(n):
    r0 = (-(-n // NS) + 15) // 16 * 16
    rlast = n - (NS - 1) * r0
    assert 0 < rlast <= r0
    return r0, rlast


def _tilewise_copy(s, n, copy_fn):
    r0, rlast = _row_split(n)

    @pl.when(s < NS - 1)
    def _():
        copy_fn(s * r0, r0)

    @pl.when(s == NS - 1)
    def _():
        copy_fn((NS - 1) * r0, rlast)


def _make_sc_deg(n, nb):
    @functools.partial(
        pl.kernel,
        mesh=_sc_mesh(),
        out_type=jax.ShapeDtypeStruct((NC, n, 16), F32),
        compiler_params=pltpu.CompilerParams(use_tc_tiling_on_sc=False),
        scratch_types=[
            pltpu.VMEM((nb, EB), jnp.int32),
            pltpu.VMEM((EB, 16), F32),
            pltpu.VMEM_SHARED((n, 16), F32),
        ],
    )
    def sc_deg(dst_hbm, ones_hbm, zeros_hbm, degp_hbm, idx_v, ones_v, dacc):
        c = lax.axis_index("c")
        s = lax.axis_index("s")
        w = c * NS + s
        pltpu.sync_copy(dst_hbm.at[w], idx_v)
        pltpu.sync_copy(ones_hbm, ones_v)
        _tilewise_copy(s, n, lambda off, sz: pltpu.sync_copy(
            zeros_hbm.at[pl.ds(0, sz)], dacc.at[pl.ds(off, sz)]))
        plsc.subcore_barrier()

        def body(j, carry):
            pltpu.sync_copy(ones_v, dacc.at[idx_v.at[j]], add=True)
            return carry

        lax.fori_loop(0, nb, body, 0)
        plsc.subcore_barrier()
        _tilewise_copy(s, n, lambda off, sz: pltpu.sync_copy(
            dacc.at[pl.ds(off, sz)], degp_hbm.at[c, pl.ds(off, sz)]))

    return sc_deg


def _make_sc_agg(n, d, nb):
    assert nb % KB == 0 and nb >= 2 * KB

    @functools.partial(
        pl.kernel,
        mesh=_sc_mesh(),
        out_type=jax.ShapeDtypeStruct((NC, n, d), BF16),
        compiler_params=pltpu.CompilerParams(use_tc_tiling_on_sc=False),
        scratch_types=(
            [pltpu.VMEM((nb, EB), jnp.int32),
             pltpu.VMEM((nb, EB), jnp.int32)]
            + [pltpu.VMEM((EB, d), BF16) for _ in range(KB)]
            + [pltpu.VMEM_SHARED((n, d), BF16)]
            + [pltpu.SemaphoreType.DMA for _ in range(KB)]
        ),
    )
    def sc_agg(yw_hbm, src_hbm, dst_hbm, out_hbm, isrc, idst, *rest):
        rows = rest[:KB]
        acc = rest[KB]
        sems = rest[KB + 1:]
        c = lax.axis_index("c")
        s = lax.axis_index("s")
        w = c * NS + s
        pltpu.sync_copy(src_hbm.at[w], isrc)
        pltpu.sync_copy(dst_hbm.at[w], idst)
        for b in range(KB):
            pltpu.async_copy(yw_hbm.at[isrc.at[b]], rows[b], sems[b])
        _tilewise_copy(s, n, lambda off, sz: pltpu.sync_copy(
            yw_hbm.at[pl.ds(off, sz)], acc.at[pl.ds(off, sz)]))
        plsc.subcore_barrier()

        def body(t, carry):
            for b in range(KB):
                j = KB * t + b
                jn = lax.min(j + KB, nb - 1)
                pltpu.make_async_copy(
                    yw_hbm.at[isrc.at[j]], rows[b], sems[b]).wait()
                pltpu.sync_copy(rows[b], acc.at[idst.at[j]], add=True)
                pltpu.async_copy(yw_hbm.at[isrc.at[jn]], rows[b], sems[b])
            return carry

        lax.fori_loop(0, nb // KB, body, 0)
        for b in range(KB):
            pltpu.make_async_copy(yw_hbm.at[isrc.at[0]], rows[b],
                                  sems[b]).wait()
        plsc.subcore_barrier()
        _tilewise_copy(s, n, lambda off, sz: pltpu.sync_copy(
            acc.at[pl.ds(off, sz)], out_hbm.at[c, pl.ds(off, sz)]))

    return sc_agg


def _row_mask(n, npad, width):
    rows = lax.broadcasted_iota(jnp.int32, (n, width), 0)
    return jnp.where(rows < npad, 1.0, 0.0).astype(F32)


def _make_tc1(n, npad):
    def _tc1_body(x_ref, w1_ref, degp_ref, yw_ref, dinv_ref):
        deg = degp_ref[0] + degp_ref[1] + (1.0 - _row_mask(n, npad, 16))
        dinv = lax.rsqrt(deg)
        dinv_ref[...] = dinv
        xw = jnp.dot(x_ref[...], w1_ref[...], preferred_element_type=F32)
        yw_ref[...] = (xw * dinv[:, 0:1]).astype(BF16)

    return _tc1_body


def _make_tc2(n, npad):
    def _tc2_body(p_ref, yw_ref, dinv_ref, gamma_ref, beta_ref, b1_ref,
                  w2_ref, zw_ref):
        dinv1 = dinv_ref[:, 0:1]
        ywc = yw_ref[...].astype(F32) * (1.0 + _row_mask(n, npad, 1))
        p = p_ref[0].astype(F32) + p_ref[1].astype(F32)
        h = dinv1 * (p - ywc) + b1_ref[...]
        mean = jnp.mean(h, axis=0, keepdims=True)
        var = jnp.mean((h - mean) ** 2, axis=0, keepdims=True)
        hn = (h - mean) * lax.rsqrt(var + EPS) * gamma_ref[...] + beta_ref[...]
        hrelu = jnp.maximum(hn, 0.0)
        hw = jnp.dot(hrelu, w2_ref[...], preferred_element_type=F32)
        zw_ref[...] = (hw * dinv1).astype(BF16)

    return _tc2_body


def _make_tc3(n, npad):
    def _tc3_body(p_ref, zw_ref, dinv_ref, b2_ref, out_ref):
        dinv1 = dinv_ref[:, 0:1]
        zwc = zw_ref[...].astype(F32) * (1.0 + _row_mask(n, npad, 1))
        p = p_ref[0].astype(F32) + p_ref[1].astype(F32)
        out_ref[...] = dinv1 * (p - zwc) + b2_ref[...]

    return _tc3_body


def kernel(x, edge_index, W1, b1, gamma, beta, W2, b2):
    n, _ = x.shape
    e = edge_index.shape[1]
    d1 = W1.shape[1]
    d2 = W2.shape[1]

    nb = -(-e // (NW * EB))
    nb = -(-nb // KB) * KB
    ep = nb * NW * EB
    npad = ep - e
    assert npad <= n

    pad = jnp.arange(npad, dtype=edge_index.dtype)
    ei = jnp.concatenate([edge_index, jnp.stack([pad, pad])], axis=1)
    src3 = ei[0].reshape(NW, nb, EB)
    dst3 = ei[1].reshape(NW, nb, EB)
    ones16 = jnp.ones((EB, 16), F32)
    zeros16 = jnp.zeros((_row_split(n)[0], 16), F32)

    degp = _make_sc_deg(n, nb)(dst3, ones16, zeros16)

    yw, dinv = pl.pallas_call(
        _make_tc1(n, npad),
        out_shape=(jax.ShapeDtypeStruct((n, d1), BF16),
                   jax.ShapeDtypeStruct((n, 16), F32)),
    )(x, W1, degp)

    p1 = _make_sc_agg(n, d1, nb)(yw, src3, dst3)

    zw = pl.pallas_call(
        _make_tc2(n, npad),
        out_shape=jax.ShapeDtypeStruct((n, d2), BF16),
    )(p1, yw, dinv, gamma.reshape(1, d1), beta.reshape(1, d1),
      b1.reshape(1, d1), W2)

    p2 = _make_sc_agg(n, d2, nb)(zw, src3, dst3)

    out = pl.pallas_call(
        _make_tc3(n, npad),
        out_shape=jax.ShapeDtypeStruct((n, d2), F32),
    )(p2, zw, dinv, b2.reshape(1, d2))

    return out


# speedup vs baseline: 1.2172x; 1.0036x over previous
import functools

import jax
import jax.numpy as jnp
from jax import lax
from jax.experimental import pallas as pl
from jax.experimental.pallas import tpu as pltpu
from jax.experimental.pallas import tpu_sc as plsc

F32 = jnp.float32
BF16 = jnp.bfloat16
EPS = 1e-5

NC = 2
NS = 16
NW = NC * NS
EB = 128
KB = 8


def _sc_mesh():
    return plsc.VectorSubcoreMesh(core_axis_name="c", subcore_axis_name="s")


def _row_split(n):
    r0 = (-(-n // NS) + 15) // 16 * 16
    rlast = n - (NS - 1) * r0
    assert 0 < rlast <= r0
    return r0, rlast


def _tilewise_copy(s, n, copy_fn):
    r0, rlast = _row_split(n)

    @pl.when(s < NS - 1)
    def _():
        copy_fn(s * r0, r0)

    @pl.when(s == NS - 1)
    def _():
        copy_fn((NS - 1) * r0, rlast)


def _make_sc_deg(n, nb):
    @functools.partial(
        pl.kernel,
        mesh=_sc_mesh(),
        out_type=jax.ShapeDtypeStruct((NC, n, 16), F32),
        compiler_params=pltpu.CompilerParams(use_tc_tiling_on_sc=False),
        scratch_types=[
            pltpu.VMEM((nb, EB), jnp.int32),
            pltpu.VMEM((EB, 16), F32),
            pltpu.VMEM_SHARED((n, 16), F32),
        ],
    )
    def sc_deg(dst_hbm, ones_hbm, zeros_hbm, degp_hbm, idx_v, ones_v, dacc):
        c = lax.axis_index("c")
        s = lax.axis_index("s")
        w = c * NS + s
        pltpu.sync_copy(dst_hbm.at[w], idx_v)
        pltpu.sync_copy(ones_hbm, ones_v)
        _tilewise_copy(s, n, lambda off, sz: pltpu.sync_copy(
            zeros_hbm.at[pl.ds(0, sz)], dacc.at[pl.ds(off, sz)]))
        plsc.subcore_barrier()

        def body(j, carry):
            pltpu.sync_copy(ones_v, dacc.at[idx_v.at[j]], add=True)
            return carry

        lax.fori_loop(0, nb, body, 0)
        plsc.subcore_barrier()
        _tilewise_copy(s, n, lambda off, sz: pltpu.sync_copy(
            dacc.at[pl.ds(off, sz)], degp_hbm.at[c, pl.ds(off, sz)]))

    return sc_deg


def _make_sc_agg(n, d, nb):
    assert nb % KB == 0 and nb >= 2 * KB

    @functools.partial(
        pl.kernel,
        mesh=_sc_mesh(),
        out_type=jax.ShapeDtypeStruct((NC, n, d), BF16),
        compiler_params=pltpu.CompilerParams(use_tc_tiling_on_sc=False),
        scratch_types=(
            [pltpu.VMEM((nb, EB), jnp.int32),
             pltpu.VMEM((nb, EB), jnp.int32)]
            + [pltpu.VMEM((EB, d), BF16) for _ in range(KB)]
            + [pltpu.VMEM_SHARED((n, d), BF16)]
            + [pltpu.SemaphoreType.DMA for _ in range(KB)]
        ),
    )
    def sc_agg(yw_hbm, src_hbm, dst_hbm, out_hbm, isrc, idst, *rest):
        rows = rest[:KB]
        acc = rest[KB]
        sems = rest[KB + 1:]
        c = lax.axis_index("c")
        s = lax.axis_index("s")
        w = c * NS + s
        pltpu.sync_copy(src_hbm.at[w], isrc)
        pltpu.sync_copy(dst_hbm.at[w], idst)
        for b in range(KB):
            pltpu.async_copy(yw_hbm.at[isrc.at[b]], rows[b], sems[b])
        _tilewise_copy(s, n, lambda off, sz: pltpu.sync_copy(
            yw_hbm.at[pl.ds(off, sz)], acc.at[pl.ds(off, sz)]))
        plsc.subcore_barrier()

        def body(t, carry):
            for b in range(KB):
                j = KB * t + b
                jn = lax.min(j + KB, nb - 1)
                pltpu.make_async_copy(
                    yw_hbm.at[isrc.at[j]], rows[b], sems[b]).wait()
                pltpu.sync_copy(rows[b], acc.at[idst.at[j]], add=True)
                pltpu.async_copy(yw_hbm.at[isrc.at[jn]], rows[b], sems[b])
            return carry

        lax.fori_loop(0, nb // KB, body, 0)
        for b in range(KB):
            pltpu.make_async_copy(yw_hbm.at[isrc.at[0]], rows[b],
                                  sems[b]).wait()
        plsc.subcore_barrier()
        _tilewise_copy(s, n, lambda off, sz: pltpu.sync_copy(
            acc.at[pl.ds(off, sz)], out_hbm.at[c, pl.ds(off, sz)]))

    return sc_agg


def _row_mask(n, npad, width):
    rows = lax.broadcasted_iota(jnp.int32, (n, width), 0)
    return jnp.where(rows < npad, 1.0, 0.0).astype(F32)


def _make_tc1(n, npad):
    def _tc1_body(x_ref, w1_ref, degp_ref, yw_ref, dinv_ref):
        deg = degp_ref[0] + degp_ref[1] + (1.0 - _row_mask(n, npad, 16))
        dinv = lax.rsqrt(deg)
        dinv_ref[...] = dinv
        xw = jnp.dot(x_ref[...], w1_ref[...], preferred_element_type=F32)
        yw_ref[...] = (xw * dinv[:, 0:1]).astype(BF16)

    return _tc1_body


def _make_tc2(n, npad):
    def _tc2_body(p_ref, yw_ref, dinv_ref, gamma_ref, beta_ref, b1_ref,
                  w2_ref, zw_ref):
        dinv1 = dinv_ref[:, 0:1]
        ywc = yw_ref[...].astype(F32) * (1.0 + _row_mask(n, npad, 1))
        p = p_ref[0].astype(F32) + p_ref[1].astype(F32)
        h = dinv1 * (p - ywc) + b1_ref[...]
        mean = jnp.mean(h, axis=0, keepdims=True)
        var = jnp.mean((h - mean) ** 2, axis=0, keepdims=True)
        hn = (h - mean) * lax.rsqrt(var + EPS) * gamma_ref[...] + beta_ref[...]
        hrelu = jnp.maximum(hn, 0.0)
        hw = jnp.dot(hrelu, w2_ref[...], preferred_element_type=F32)
        zw_ref[...] = (hw * dinv1).astype(BF16)

    return _tc2_body


def _make_tc3(n, npad):
    def _tc3_body(p_ref, zw_ref, dinv_ref, b2_ref, out_ref):
        dinv1 = dinv_ref[:, 0:1]
        zwc = zw_ref[...].astype(F32) * (1.0 + _row_mask(n, npad, 1))
        p = p_ref[0].astype(F32) + p_ref[1].astype(F32)
        out_ref[...] = dinv1 * (p - zwc) + b2_ref[...]

    return _tc3_body


def kernel(x, edge_index, W1, b1, gamma, beta, W2, b2):
    n, _ = x.shape
    e = edge_index.shape[1]
    d1 = W1.shape[1]
    d2 = W2.shape[1]

    nb = -(-e // (NW * EB))
    nb = -(-nb // KB) * KB
    ep = nb * NW * EB
    npad = ep - e
    assert npad <= n

    pad = jnp.arange(npad, dtype=edge_index.dtype)
    ei = jnp.concatenate([edge_index, jnp.stack([pad, pad])], axis=1)
    src3 = ei[0].reshape(NW, nb, EB)
    dst3 = ei[1].reshape(NW, nb, EB)
    ones16 = jnp.ones((EB, 16), F32)
    zeros16 = jnp.zeros((_row_split(n)[0], 16), F32)

    degp = _make_sc_deg(n, nb)(dst3, ones16, zeros16)

    yw, dinv = pl.pallas_call(
        _make_tc1(n, npad),
        out_shape=(jax.ShapeDtypeStruct((n, d1), BF16),
                   jax.ShapeDtypeStruct((n, 16), F32)),
    )(x, W1, degp)

    p1 = _make_sc_agg(n, d1, nb)(yw, src3, dst3)

    zw = pl.pallas_call(
        _make_tc2(n, npad),
        out_shape=jax.ShapeDtypeStruct((n, d2), BF16),
    )(p1, yw, dinv, gamma.reshape(1, d1), beta.reshape(1, d1),
      b1.reshape(1, d1), W2)

    p2 = _make_sc_agg(n, d2, nb)(zw, src3, dst3)

    out = pl.pallas_call(
        _make_tc3(n, npad),
        out_shape=jax.ShapeDtypeStruct((n, d2), F32),
    )(p2, zw, dinv, b2.reshape(1, d2))

    return out
